# baseline (speedup 1.0000x reference)
import functools

import jax
import jax.numpy as jnp
from jax import lax
from jax.experimental import pallas as pl
from jax.experimental.pallas import tpu as pltpu
from jax.experimental.pallas import tpu_sc as plsc

N = 10000
H = 128
E = 320000
NPAD = 10240
TRASH = N
NCORES = 2
NSUB = 16
NWORK = NCORES * NSUB
EPW = E // NWORK
K = 80
NCHUNK = EPW // K
KS = 40
NCHUNKS = EPW // KS
NBUF = 5
RPT = NPAD // NSUB
EPS = 1e-5

def _load_cols_2d(col_hbm, base, col2d, sem, k, nchunk):
    def issue(j, carry):
        pltpu.async_copy(col_hbm.at[pl.ds(base + j * k, k)], col2d.at[j], sem)
        return carry

    lax.fori_loop(0, nchunk, issue, 0)

    def drain(j, carry):
        pltpu.make_async_copy(col_hbm.at[pl.ds(base + j * k, k)],
                              col2d.at[j], sem).wait()
        return carry

    lax.fori_loop(0, nchunk, drain, 0)


def _sc_mesh():
    return plsc.VectorSubcoreMesh(core_axis_name="c", subcore_axis_name="s",
                                  num_cores=NCORES, num_subcores=NSUB)


@functools.cache
def _get_deg_kernel():
    return pl.kernel(
        _deg_body,
        out_type=jax.ShapeDtypeStruct((NCORES, NPAD, H), jnp.float32),
        mesh=_sc_mesh(),
        scratch_types=[
            pltpu.VMEM((NCHUNK, K), jnp.int32),
            pltpu.VMEM((K, H), jnp.float32),
            pltpu.VMEM((8, H), jnp.float32),
            pltpu.VMEM_SHARED((NPAD, H), jnp.float32),
            pltpu.SemaphoreType.DMA,
        ],
    )


def _deg_body(colp_hbm, onehot_hbm, zeros_hbm, out_hbm, col2d, ones_v, zbuf,
              acc, sem):
    cid = lax.axis_index("c")
    sid = lax.axis_index("s")
    base = (cid * NSUB + sid) * EPW

    pltpu.async_copy(onehot_hbm, ones_v, sem)
    pltpu.sync_copy(zeros_hbm, zbuf)

    def zero_acc(j, carry):
        pltpu.sync_copy(zbuf, acc.at[pl.ds(sid * RPT + j * 8, 8)])
        return carry

    lax.fori_loop(0, RPT // 8, zero_acc, 0)
    pltpu.make_async_copy(onehot_hbm, ones_v, sem).wait()
    _load_cols_2d(colp_hbm, base, col2d, sem, K, NCHUNK)
    plsc.subcore_barrier()

    def body(j, carry):
        pltpu.async_copy(ones_v, acc.at[col2d.at[j]], sem, add=True)
        return carry

    lax.fori_loop(0, NCHUNK, body, 0)

    def drain(j, carry):
        pltpu.make_async_copy(ones_v, acc.at[col2d.at[j]], sem).wait()
        return carry

    lax.fori_loop(0, NCHUNK, drain, 0)
    plsc.subcore_barrier()
    pltpu.sync_copy(acc.at[pl.ds(sid * RPT, RPT)],
                    out_hbm.at[cid, pl.ds(sid * RPT, RPT)])


@functools.cache
def _get_scatter_kernel():
    return pl.kernel(
        _scatter_body,
        out_type=jax.ShapeDtypeStruct((NCORES, NPAD, H), jnp.float32),
        mesh=_sc_mesh(),
        scratch_types=[
            pltpu.VMEM((EPW,), jnp.int32),
            pltpu.VMEM((EPW,), jnp.int32),
            pltpu.VMEM((NBUF, KS, H), jnp.float32),
            pltpu.VMEM((8, H), jnp.float32),
            pltpu.VMEM_SHARED((NPAD, H), jnp.float32),
        ] + [pltpu.SemaphoreType.DMA] * (2 * NBUF),
    )


def _scatter_body(g_hbm, row_hbm, colp_hbm, zeros_hbm, out_hbm,
                  row_all, colp_all, bufs, zbuf, acc, *sems):
    gsem = sems[:NBUF]
    ssem = sems[NBUF:]
    cid = lax.axis_index("c")
    sid = lax.axis_index("s")
    base = (cid * NSUB + sid) * EPW
    pltpu.async_copy(row_hbm.at[pl.ds(base, EPW)], row_all, gsem[0])
    pltpu.async_copy(colp_hbm.at[pl.ds(base, EPW)], colp_all, gsem[1])

    pltpu.sync_copy(zeros_hbm, zbuf)

    def zero_acc(j, carry):
        pltpu.sync_copy(zbuf, acc.at[pl.ds(sid * RPT + j * 8, 8)])
        return carry

    lax.fori_loop(0, RPT // 8, zero_acc, 0)
    pltpu.make_async_copy(row_hbm.at[pl.ds(base, EPW)], row_all,
                          gsem[0]).wait()
    pltpu.make_async_copy(colp_hbm.at[pl.ds(base, EPW)], colp_all,
                          gsem[1]).wait()
    plsc.subcore_barrier()

    def gather(c, b):
        pltpu.async_copy(g_hbm.at[row_all.at[pl.ds(c * KS, KS)]],
                         bufs.at[b], gsem[b])

    def gather_wait(c, b):
        pltpu.make_async_copy(g_hbm.at[row_all.at[pl.ds(c * KS, KS)]],
                              bufs.at[b], gsem[b]).wait()

    def scatter(c, b):
        pltpu.async_copy(bufs.at[b], acc.at[colp_all.at[pl.ds(c * KS, KS)]],
                         ssem[b], add=True)

    def scatter_wait(c, b):
        pltpu.make_async_copy(bufs.at[b],
                              acc.at[colp_all.at[pl.ds(c * KS, KS)]],
                              ssem[b]).wait()

    for b in range(NBUF):
        gather(b, b)

    def body(m, carry):
        c = m * NBUF
        for b in range(NBUF):
            gather_wait(c + b, b)
            scatter(c + b, b)
        for b in range(NBUF):
            scatter_wait(c + b, b)
            gather(c + NBUF + b, b)
        return carry

    lax.fori_loop(0, NCHUNKS // NBUF - 1, body, 0)
    cl = NCHUNKS - NBUF
    for b in range(NBUF):
        gather_wait(cl + b, b)
        scatter(cl + b, b)
    for b in range(NBUF):
        scatter_wait(cl + b, b)

    plsc.subcore_barrier()
    pltpu.sync_copy(acc.at[pl.ds(sid * RPT, RPT)],
                    out_hbm.at[cid, pl.ds(sid * RPT, RPT)])


BN = 1000
GRID = N // BN

_prec = lax.Precision.HIGHEST


def _pre_body(row_ref, col_ref, x_ref, w1_ref, b1_ref, colp_ref, h_ref):
    @pl.when(pl.program_id(0) == 0)
    def _():
        colp_ref[...] = jnp.where(row_ref[...] == col_ref[...], TRASH,
                                  col_ref[...])

    h_ref[...] = jnp.dot(x_ref[...], w1_ref[...], precision=_prec,
                         preferred_element_type=jnp.float32) + b1_ref[0]


def _pre(row, col, x, W1, b1):
    r2 = row.reshape(E // 128, 128)
    c2 = col.reshape(E // 128, 128)
    colp, h1 = pl.pallas_call(
        _pre_body,
        grid=(5,),
        in_specs=[
            pl.BlockSpec((E // 128, 128), lambda i: (0, 0)),
            pl.BlockSpec((E // 128, 128), lambda i: (0, 0)),
            pl.BlockSpec((N // 5, H), lambda i: (i, 0)),
            pl.BlockSpec((H, H), lambda i: (0, 0)),
            pl.BlockSpec((1, H), lambda i: (0, 0)),
        ],
        out_specs=[
            pl.BlockSpec((E // 128, 128), lambda i: (0, 0)),
            pl.BlockSpec((N // 5, H), lambda i: (i, 0)),
        ],
        out_shape=[
            jax.ShapeDtypeStruct((E // 128, 128), jnp.int32),
            jax.ShapeDtypeStruct((N, H), jnp.float32),
        ],
    )(r2, c2, x, W1, b1)
    return colp.reshape(E), h1


def _scale_body(h_ref, dega_ref, g_ref, dinv_ref):
    deg = (jnp.sum(dega_ref[0], axis=-1) + jnp.sum(dega_ref[1], axis=-1)
           + 1.0)
    dinv = lax.rsqrt(deg)
    g_ref[...] = h_ref[...] * dinv[:, None]
    dinv_ref[...] = dinv[:, None]


def _scale(h1, dega):
    return pl.pallas_call(
        _scale_body,
        grid=(GRID,),
        in_specs=[
            pl.BlockSpec((BN, H), lambda i: (i, 0)),
            pl.BlockSpec((NCORES, BN, H), lambda i: (0, i, 0)),
        ],
        out_specs=[
            pl.BlockSpec((BN, H), lambda i: (i, 0)),
            pl.BlockSpec((BN, 1), lambda i: (i, 0)),
        ],
        out_shape=[
            jax.ShapeDtypeStruct((N, H), jnp.float32),
            jax.ShapeDtypeStruct((N, 1), jnp.float32),
        ],
    )(h1, dega[:, :N, :])


def _bnmm_body(s_ref, g_ref, dinv_ref, gamma_ref, beta_ref, w2_ref, b2_ref,
               g2_ref, stats_ref):
    ph = pl.program_id(0)
    i = pl.program_id(1)
    p = (s_ref[0] + s_ref[1] + g_ref[...]) * dinv_ref[...]

    @pl.when(ph == 0)
    def _():
        new = jnp.stack([jnp.sum(p, axis=0), jnp.sum(p * p, axis=0)])

        @pl.when(i == 0)
        def _():
            stats_ref[...] = new

        @pl.when(i > 0)
        def _():
            stats_ref[...] = stats_ref[...] + new

    @pl.when(ph == 1)
    def _():
        mean = stats_ref[0] / N
        var = stats_ref[1] / N - mean * mean
        inv = lax.rsqrt(var + EPS)
        hn = (p - mean) * (inv * gamma_ref[0]) + beta_ref[0]
        hn = jnp.maximum(hn, 0.0)
        h2 = jnp.dot(hn, w2_ref[...], precision=_prec,
                     preferred_element_type=jnp.float32) + b2_ref[0]
        g2_ref[...] = h2 * dinv_ref[...]


def _bnmm(s, g, dinv, gamma, beta, W2, b2):
    out = pl.pallas_call(
        _bnmm_body,
        grid=(2, GRID),
        in_specs=[
            pl.BlockSpec((NCORES, BN, H), lambda p, i: (0, i, 0)),
            pl.BlockSpec((BN, H), lambda p, i: (i, 0)),
            pl.BlockSpec((BN, 1), lambda p, i: (i, 0)),
            pl.BlockSpec((1, H), lambda p, i: (0, 0)),
            pl.BlockSpec((1, H), lambda p, i: (0, 0)),
            pl.BlockSpec((H, H), lambda p, i: (0, 0)),
            pl.BlockSpec((1, H), lambda p, i: (0, 0)),
        ],
        out_specs=pl.BlockSpec((BN, H),
                               lambda p, i: (jnp.where(p == 0, GRID, i), 0)),
        out_shape=jax.ShapeDtypeStruct((N + BN, H), jnp.float32),
        scratch_shapes=[pltpu.VMEM((2, H), jnp.float32)],
    )(s[:, :N, :], g, dinv, gamma, beta, W2, b2)
    return out[:N]


def _final_body(s_ref, g2_ref, dinv_ref, o_ref):
    p = (s_ref[0] + s_ref[1] + g2_ref[...]) * dinv_ref[...]
    m = jnp.max(p, axis=1, keepdims=True)
    lse = jnp.log(jnp.sum(jnp.exp(p - m), axis=1, keepdims=True)) + m
    o_ref[...] = p - lse


def _final(s, g2, dinv):
    return pl.pallas_call(
        _final_body,
        grid=(GRID,),
        in_specs=[
            pl.BlockSpec((NCORES, BN, H), lambda i: (0, i, 0)),
            pl.BlockSpec((BN, H), lambda i: (i, 0)),
            pl.BlockSpec((BN, 1), lambda i: (i, 0)),
        ],
        out_specs=pl.BlockSpec((BN, H), lambda i: (i, 0)),
        out_shape=jax.ShapeDtypeStruct((N, H), jnp.float32),
    )(s[:, :N, :], g2, dinv)


def kernel(x, edge_index, W1, b1, gamma, beta, W2, b2):
    row = edge_index[0].astype(jnp.int32)
    col = edge_index[1].astype(jnp.int32)
    b1 = b1.reshape(1, H)
    b2 = b2.reshape(1, H)
    gamma = gamma.reshape(1, H)
    beta = beta.reshape(1, H)

    onehot = jnp.zeros((K, H), jnp.float32).at[:, 0].set(1.0)
    zerosh = jnp.zeros((8, H), jnp.float32)

    colp, h1 = _pre(row, col, x, W1, b1)
    dega = _get_deg_kernel()(colp, onehot, zerosh)
    g1, dinv = _scale(h1, dega)
    s1 = _get_scatter_kernel()(g1, row, colp, zerosh)
    g2 = _bnmm(s1, g1, dinv, gamma, beta, W2, b2)
    s2 = _get_scatter_kernel()(g2, row, colp, zerosh)
    return _final(s2, g2, dinv)


# speedup vs baseline: 1.0260x; 1.0260x over previous
import functools

import jax
import jax.numpy as jnp
from jax import lax
from jax.experimental import pallas as pl
from jax.experimental.pallas import tpu as pltpu
from jax.experimental.pallas import tpu_sc as plsc

N = 10000
H = 128
E = 320000
NPAD = 10240
TRASH = N
NCORES = 2
NSUB = 16
NWORK = NCORES * NSUB
EPW = E // NWORK
K = 80
NCHUNK = EPW // K
KS = 40
NCHUNKS = EPW // KS
NBUF = 5
RPT = NPAD // NSUB
EPS = 1e-5

def _load_cols_2d(col_hbm, base, col2d, sem, k, nchunk):
    def issue(j, carry):
        pltpu.async_copy(col_hbm.at[pl.ds(base + j * k, k)], col2d.at[j], sem)
        return carry

    lax.fori_loop(0, nchunk, issue, 0)

    def drain(j, carry):
        pltpu.make_async_copy(col_hbm.at[pl.ds(base + j * k, k)],
                              col2d.at[j], sem).wait()
        return carry

    lax.fori_loop(0, nchunk, drain, 0)


def _sc_mesh():
    return plsc.VectorSubcoreMesh(core_axis_name="c", subcore_axis_name="s",
                                  num_cores=NCORES, num_subcores=NSUB)


@functools.cache
def _get_deg_kernel():
    return pl.kernel(
        _deg_body,
        out_type=jax.ShapeDtypeStruct((NCORES, NPAD, H), jnp.float32),
        mesh=_sc_mesh(),
        scratch_types=[
            pltpu.VMEM((NCHUNK, K), jnp.int32),
            pltpu.VMEM((K, H), jnp.float32),
            pltpu.VMEM((8, H), jnp.float32),
            pltpu.VMEM_SHARED((NPAD, H), jnp.float32),
            pltpu.SemaphoreType.DMA,
        ],
    )


def _deg_body(colp_hbm, onehot_hbm, zeros_hbm, out_hbm, col2d, ones_v, zbuf,
              acc, sem):
    cid = lax.axis_index("c")
    sid = lax.axis_index("s")
    base = (cid * NSUB + sid) * EPW

    pltpu.async_copy(onehot_hbm, ones_v, sem)
    pltpu.sync_copy(zeros_hbm, zbuf)

    def zero_acc(j, carry):
        pltpu.sync_copy(zbuf, acc.at[pl.ds(sid * RPT + j * 8, 8)])
        return carry

    lax.fori_loop(0, RPT // 8, zero_acc, 0)
    pltpu.make_async_copy(onehot_hbm, ones_v, sem).wait()
    _load_cols_2d(colp_hbm, base, col2d, sem, K, NCHUNK)
    plsc.subcore_barrier()

    def body(j, carry):
        pltpu.async_copy(ones_v, acc.at[col2d.at[j]], sem, add=True)
        return carry

    lax.fori_loop(0, NCHUNK, body, 0)

    def drain(j, carry):
        pltpu.make_async_copy(ones_v, acc.at[col2d.at[j]], sem).wait()
        return carry

    lax.fori_loop(0, NCHUNK, drain, 0)
    plsc.subcore_barrier()
    pltpu.sync_copy(acc.at[pl.ds(sid * RPT, RPT)],
                    out_hbm.at[cid, pl.ds(sid * RPT, RPT)])


@functools.cache
def _get_scatter_kernel():
    return pl.kernel(
        _scatter_body,
        out_type=jax.ShapeDtypeStruct((NCORES, NPAD, H), jnp.float32),
        mesh=_sc_mesh(),
        scratch_types=[
            pltpu.VMEM((EPW,), jnp.int32),
            pltpu.VMEM((EPW,), jnp.int32),
            pltpu.VMEM((NBUF, KS, H), jnp.float32),
            pltpu.VMEM((8, H), jnp.float32),
            pltpu.VMEM_SHARED((NPAD, H), jnp.float32),
        ] + [pltpu.SemaphoreType.DMA] * (2 * NBUF),
    )


def _scatter_body(g_hbm, row_hbm, colp_hbm, zeros_hbm, out_hbm,
                  row_all, colp_all, bufs, zbuf, acc, *sems):
    gsem = sems[:NBUF]
    ssem = sems[NBUF:]
    cid = lax.axis_index("c")
    sid = lax.axis_index("s")
    base = (cid * NSUB + sid) * EPW
    pltpu.async_copy(row_hbm.at[pl.ds(base, EPW)], row_all, gsem[0])
    pltpu.async_copy(colp_hbm.at[pl.ds(base, EPW)], colp_all, gsem[1])

    pltpu.sync_copy(zeros_hbm, zbuf)

    def zero_acc(j, carry):
        pltpu.sync_copy(zbuf, acc.at[pl.ds(sid * RPT + j * 8, 8)])
        return carry

    lax.fori_loop(0, RPT // 8, zero_acc, 0)
    pltpu.make_async_copy(row_hbm.at[pl.ds(base, EPW)], row_all,
                          gsem[0]).wait()
    pltpu.make_async_copy(colp_hbm.at[pl.ds(base, EPW)], colp_all,
                          gsem[1]).wait()
    plsc.subcore_barrier()

    def gather(c, b):
        pltpu.async_copy(g_hbm.at[row_all.at[pl.ds(c * KS, KS)]],
                         bufs.at[b], gsem[b])

    def gather_wait(c, b):
        pltpu.make_async_copy(g_hbm.at[row_all.at[pl.ds(c * KS, KS)]],
                              bufs.at[b], gsem[b]).wait()

    def scatter(c, b):
        pltpu.async_copy(bufs.at[b], acc.at[colp_all.at[pl.ds(c * KS, KS)]],
                         ssem[b], add=True)

    def scatter_wait(c, b):
        pltpu.make_async_copy(bufs.at[b],
                              acc.at[colp_all.at[pl.ds(c * KS, KS)]],
                              ssem[b]).wait()

    for b in range(NBUF):
        gather(b, b)

    def body(m, carry):
        c = m * NBUF
        for b in range(NBUF):
            gather_wait(c + b, b)
            scatter(c + b, b)
        for b in range(NBUF):
            scatter_wait(c + b, b)
            gather(c + NBUF + b, b)
        return carry

    lax.fori_loop(0, NCHUNKS // NBUF - 1, body, 0)
    cl = NCHUNKS - NBUF
    for b in range(NBUF):
        gather_wait(cl + b, b)
        scatter(cl + b, b)
    for b in range(NBUF):
        scatter_wait(cl + b, b)

    plsc.subcore_barrier()
    pltpu.sync_copy(acc.at[pl.ds(sid * RPT, RPT)],
                    out_hbm.at[cid, pl.ds(sid * RPT, RPT)])


BN = 1000
GRID = N // BN

_prec = lax.Precision.HIGHEST


def _colfix_body(row_ref, col_ref, out_ref):
    out_ref[...] = jnp.where(row_ref[...] == col_ref[...], TRASH, col_ref[...])


def _colfix(row, col):
    r2 = row.reshape(E // 128, 128)
    c2 = col.reshape(E // 128, 128)
    out = pl.pallas_call(
        _colfix_body,
        out_shape=jax.ShapeDtypeStruct((E // 128, 128), jnp.int32),
    )(r2, c2)
    return out.reshape(E)


def _mm1_body(x_ref, w1_ref, b1_ref, h_ref):
    h_ref[...] = jnp.dot(x_ref[...], w1_ref[...], precision=_prec,
                         preferred_element_type=jnp.float32) + b1_ref[0]


def _mm1(x, W1, b1):
    return pl.pallas_call(
        _mm1_body,
        grid=(GRID,),
        in_specs=[
            pl.BlockSpec((BN, H), lambda i: (i, 0)),
            pl.BlockSpec((H, H), lambda i: (0, 0)),
            pl.BlockSpec((1, H), lambda i: (0, 0)),
        ],
        out_specs=pl.BlockSpec((BN, H), lambda i: (i, 0)),
        out_shape=jax.ShapeDtypeStruct((N, H), jnp.float32),
    )(x, W1, b1)


def _scale_body(h_ref, dega_ref, g_ref, dinv_ref):
    deg = (jnp.sum(dega_ref[0], axis=-1) + jnp.sum(dega_ref[1], axis=-1)
           + 1.0)
    dinv = lax.rsqrt(deg)
    g_ref[...] = h_ref[...] * dinv[:, None]
    dinv_ref[...] = dinv[:, None]


def _scale(h1, dega):
    return pl.pallas_call(
        _scale_body,
        grid=(GRID,),
        in_specs=[
            pl.BlockSpec((BN, H), lambda i: (i, 0)),
            pl.BlockSpec((NCORES, BN, H), lambda i: (0, i, 0)),
        ],
        out_specs=[
            pl.BlockSpec((BN, H), lambda i: (i, 0)),
            pl.BlockSpec((BN, 1), lambda i: (i, 0)),
        ],
        out_shape=[
            jax.ShapeDtypeStruct((N, H), jnp.float32),
            jax.ShapeDtypeStruct((N, 1), jnp.float32),
        ],
    )(h1, dega[:, :N, :])


def _stats_body(s_ref, g_ref, dinv_ref, p_ref, stats_ref):
    i = pl.program_id(0)
    p = (s_ref[0] + s_ref[1] + g_ref[...]) * dinv_ref[...]
    p_ref[...] = p
    new = jnp.stack([jnp.sum(p, axis=0), jnp.sum(p * p, axis=0)])

    @pl.when(i == 0)
    def _():
        stats_ref[...] = new

    @pl.when(i > 0)
    def _():
        stats_ref[...] = stats_ref[...] + new


def _stats(s, g, dinv):
    return pl.pallas_call(
        _stats_body,
        grid=(GRID,),
        in_specs=[
            pl.BlockSpec((NCORES, BN, H), lambda i: (0, i, 0)),
            pl.BlockSpec((BN, H), lambda i: (i, 0)),
            pl.BlockSpec((BN, 1), lambda i: (i, 0)),
        ],
        out_specs=[
            pl.BlockSpec((BN, H), lambda i: (i, 0)),
            pl.BlockSpec((2, H), lambda i: (0, 0)),
        ],
        out_shape=[
            jax.ShapeDtypeStruct((N, H), jnp.float32),
            jax.ShapeDtypeStruct((2, H), jnp.float32),
        ],
    )(s[:, :N, :], g, dinv)


def _dense2_body(p_ref, stats_ref, gamma_ref, beta_ref, w2_ref, b2_ref,
                 dinv_ref, g2_ref):
    mean = stats_ref[0] / N
    var = stats_ref[1] / N - mean * mean
    inv = lax.rsqrt(var + EPS)
    hn = (p_ref[...] - mean) * (inv * gamma_ref[0]) + beta_ref[0]
    hn = jnp.maximum(hn, 0.0)
    h2 = jnp.dot(hn, w2_ref[...], precision=_prec,
                 preferred_element_type=jnp.float32) + b2_ref[0]
    g2_ref[...] = h2 * dinv_ref[...]


def _dense2(p, stats, gamma, beta, W2, b2, dinv):
    return pl.pallas_call(
        _dense2_body,
        grid=(GRID,),
        in_specs=[
            pl.BlockSpec((BN, H), lambda i: (i, 0)),
            pl.BlockSpec((2, H), lambda i: (0, 0)),
            pl.BlockSpec((1, H), lambda i: (0, 0)),
            pl.BlockSpec((1, H), lambda i: (0, 0)),
            pl.BlockSpec((H, H), lambda i: (0, 0)),
            pl.BlockSpec((1, H), lambda i: (0, 0)),
            pl.BlockSpec((BN, 1), lambda i: (i, 0)),
        ],
        out_specs=pl.BlockSpec((BN, H), lambda i: (i, 0)),
        out_shape=jax.ShapeDtypeStruct((N, H), jnp.float32),
    )(p, stats, gamma, beta, W2, b2, dinv)


def _final_body(s_ref, g2_ref, dinv_ref, o_ref):
    p = (s_ref[0] + s_ref[1] + g2_ref[...]) * dinv_ref[...]
    m = jnp.max(p, axis=1, keepdims=True)
    lse = jnp.log(jnp.sum(jnp.exp(p - m), axis=1, keepdims=True)) + m
    o_ref[...] = p - lse


def _final(s, g2, dinv):
    return pl.pallas_call(
        _final_body,
        grid=(GRID,),
        in_specs=[
            pl.BlockSpec((NCORES, BN, H), lambda i: (0, i, 0)),
            pl.BlockSpec((BN, H), lambda i: (i, 0)),
            pl.BlockSpec((BN, 1), lambda i: (i, 0)),
        ],
        out_specs=pl.BlockSpec((BN, H), lambda i: (i, 0)),
        out_shape=jax.ShapeDtypeStruct((N, H), jnp.float32),
    )(s[:, :N, :], g2, dinv)


def kernel(x, edge_index, W1, b1, gamma, beta, W2, b2):
    row = edge_index[0].astype(jnp.int32)
    col = edge_index[1].astype(jnp.int32)
    b1 = b1.reshape(1, H)
    b2 = b2.reshape(1, H)
    gamma = gamma.reshape(1, H)
    beta = beta.reshape(1, H)

    onehot = jnp.zeros((K, H), jnp.float32).at[:, 0].set(1.0)
    zerosh = jnp.zeros((8, H), jnp.float32)

    colp = _colfix(row, col)
    dega = _get_deg_kernel()(colp, onehot, zerosh)
    h1 = _mm1(x, W1, b1)
    g1, dinv = _scale(h1, dega)
    s1 = _get_scatter_kernel()(g1, row, colp, zerosh)
    p1, stats = _stats(s1, g1, dinv)
    g2 = _dense2(p1, stats, gamma, beta, W2, b2, dinv)
    s2 = _get_scatter_kernel()(g2, row, colp, zerosh)
    return _final(s2, g2, dinv)
